# baseline (speedup 1.0000x reference)
import jax
import jax.numpy as jnp
from jax import lax
from jax.experimental import pallas as pl
from jax.experimental.pallas import tpu as pltpu
from jax.experimental.pallas import tpu_sc as plsc

NC = 2
NS = 16
NW = NC * NS
B = 16384
D = 32
BPW = B // NW
CHUNK = 128
NCHUNK = BPW // CHUNK


def _gmf_body(user_hbm, item_hbm, uw_hbm, iw_hbm, params_hbm, out_hbm,
              uidx_v, iidx_v, urows_v, irows_v, params_v, out_v, sem):
    wid = lax.axis_index("s") * NC + lax.axis_index("c")
    pltpu.sync_copy(user_hbm.at[wid], uidx_v)
    pltpu.sync_copy(item_hbm.at[wid], iidx_v)
    pltpu.sync_copy(params_hbm, params_v)

    copies = []
    for j in range(NCHUNK):
        copies.append(pltpu.async_copy(
            uw_hbm.at[uidx_v.at[j]],
            urows_v.at[pl.ds(j * CHUNK, CHUNK)], sem))
        copies.append(pltpu.async_copy(
            iw_hbm.at[iidx_v.at[j]],
            irows_v.at[pl.ds(j * CHUNK, CHUNK)], sem))
    for c in copies:
        c.wait()

    w_lo = params_v[pl.ds(0, 16)]
    w_hi = params_v[pl.ds(16, 16)]
    bias = params_v[pl.ds(32, 16)][0]
    wd = [w_lo[d] for d in range(16)] + [w_hi[d] for d in range(16)]
    lane = jnp.arange(16, dtype=jnp.int32)

    def body(g, carry):
        rows = g * 16 + lane
        acc = jnp.zeros((16,), jnp.float32)
        for d in range(D):
            col = jnp.full((16,), d, jnp.int32)
            u = plsc.load_gather(urows_v, [rows, col])
            i = plsc.load_gather(irows_v, [rows, col])
            acc = acc + (u * i) * wd[d]
        out_v[pl.ds(g * 16, 16)] = acc + bias
        return carry

    lax.fori_loop(0, BPW // 16, body, 0)
    pltpu.sync_copy(out_v, out_hbm.at[wid])


def kernel(user, item, embed_user_weight, embed_item_weight, predict_weight,
           predict_bias):
    user3 = user.reshape(NW, NCHUNK, CHUNK)
    item3 = item.reshape(NW, NCHUNK, CHUNK)
    params = jnp.concatenate([
        predict_weight.reshape(D), predict_bias,
        jnp.zeros((15,), jnp.float32)])
    mesh = plsc.VectorSubcoreMesh(core_axis_name="c", subcore_axis_name="s")
    k = pl.kernel(
        _gmf_body,
        out_type=jax.ShapeDtypeStruct((NW, BPW), jnp.float32),
        mesh=mesh,
        scratch_types=[
            pltpu.VMEM((NCHUNK, CHUNK), jnp.int32),
            pltpu.VMEM((NCHUNK, CHUNK), jnp.int32),
            pltpu.VMEM((BPW, D), jnp.float32),
            pltpu.VMEM((BPW, D), jnp.float32),
            pltpu.VMEM((48,), jnp.float32),
            pltpu.VMEM((BPW,), jnp.float32),
            pltpu.SemaphoreType.DMA,
        ],
        compiler_params=pltpu.CompilerParams(
            needs_layout_passes=False, use_tc_tiling_on_sc=False),
    )
    out = k(user3, item3, embed_user_weight, embed_item_weight, params)
    return out.reshape(B)
